# Initial kernel scaffold; baseline (speedup 1.0000x reference)
import jax
import jax.numpy as jnp
from jax.experimental import pallas as pl


def kernel(x, eidx, etype, W0, q0, k0, b0, P0, pb0, W1, q1, k1, b1, P1, pb1):
    raise NotImplementedError("write your pallas kernel here")


# speedup vs baseline: 35.0308x; 35.0308x over previous
import functools

import jax
import jax.numpy as jnp
from jax import lax
from jax.experimental import pallas as pl
from jax.experimental.pallas import tpu as pltpu
from jax.experimental.pallas import tpu_sc as plsc

N = 10000
E = 320000
DIM = 128
HEADS = 4
OUT = 32
NREL = 2
NEG = 0.2

N2 = 10240
EPAD = 327680
K = 1024
KB = K // 128
ECHUNK = EPAD // 8
NBLK = ECHUNK // K
BN = 1000


def _tc_proj(hin, W, q, k):
    grid = (NREL, N // BN)

    def body(h_ref, w_ref, q_ref, k_ref, xw_ref, qn_ref, kn_ref):
        xb = h_ref[...]
        xw = jnp.dot(xb, w_ref[0], preferred_element_type=jnp.float32)
        xw_ref[0] = xw
        qn_ref[0] = jnp.dot(xw, q_ref[...], preferred_element_type=jnp.float32)
        kn_ref[0] = jnp.dot(xw, k_ref[...], preferred_element_type=jnp.float32)

    return pl.pallas_call(
        body,
        grid=grid,
        in_specs=[
            pl.BlockSpec((BN, DIM), lambda r, nb: (nb, 0)),
            pl.BlockSpec((1, DIM, DIM), lambda r, nb: (r, 0, 0)),
            pl.BlockSpec((DIM, HEADS), lambda r, nb: (0, 0)),
            pl.BlockSpec((DIM, HEADS), lambda r, nb: (0, 0)),
        ],
        out_specs=[
            pl.BlockSpec((1, BN, DIM), lambda r, nb: (r, nb, 0)),
            pl.BlockSpec((1, BN, HEADS), lambda r, nb: (r, nb, 0)),
            pl.BlockSpec((1, BN, HEADS), lambda r, nb: (r, nb, 0)),
        ],
        out_shape=[
            jax.ShapeDtypeStruct((NREL, N, DIM), jnp.float32),
            jax.ShapeDtypeStruct((NREL, N, HEADS), jnp.float32),
            jax.ShapeDtypeStruct((NREL, N, HEADS), jnp.float32),
        ],
    )(hin, W, q, k)


def _tc_finish(aggr, den, b2, P, pb2):
    grid = (N // BN,)

    def body(a_ref, d_ref, b_ref, p_ref, pb_ref, o_ref):
        A = a_ref[...]
        d = d_ref[...]
        ag = jnp.concatenate(
            [A[h // 2, :, h % 2, :] for h in range(HEADS)], axis=1)
        den128 = jnp.concatenate(
            [jnp.broadcast_to(d[h // 2, :, h % 2:h % 2 + 1], (BN, OUT))
             for h in range(HEADS)], axis=1)
        feat = ag / (den128 + 1e-16) + b_ref[...]
        y = jnp.dot(feat, p_ref[...], preferred_element_type=jnp.float32)
        y = y + pb_ref[...]
        o_ref[...] = jnp.where(y > 0, y, jnp.exp(jnp.minimum(y, 0.0)) - 1.0)

    return pl.pallas_call(
        body,
        grid=grid,
        in_specs=[
            pl.BlockSpec((2, BN, 2, OUT), lambda nb: (0, nb, 0, 0)),
            pl.BlockSpec((2, BN, 2), lambda nb: (0, nb, 0)),
            pl.BlockSpec((1, DIM), lambda nb: (0, 0)),
            pl.BlockSpec((DIM, DIM), lambda nb: (0, 0)),
            pl.BlockSpec((1, DIM), lambda nb: (0, 0)),
        ],
        out_specs=pl.BlockSpec((BN, DIM), lambda nb: (nb, 0)),
        out_shape=jax.ShapeDtypeStruct((N, DIM), jnp.float32),
    )(aggr, den, b2, P, pb2)


def _sc_body(qidx_hbm, kidx_hbm, dst_hbm, qn_hbm, kn_hbm, xw_hbm,
             aggr_out, den_out,
             qh, kh, qib, kib, dib, gidx, sidx, exb, rows,
             zb2, zb1, aggrS, denS):
    c = lax.axis_index("c")
    s = lax.axis_index("s")
    hh = lax.rem(s, 2)
    h = 2 * c + hh
    chunk = lax.div(s, 2)

    pltpu.sync_copy(qn_hbm.at[h], qh)
    pltpu.sync_copy(kn_hbm.at[h], kh)

    zv = jnp.zeros((16,), jnp.float32)
    for jr in range(80):
        zb2[jr, pl.ds(0, 16)] = zv
        zb2[jr, pl.ds(16, 16)] = zv

    def z1(i, _):
        zb1[pl.ds(i * 16, 16)] = zv
        return 0
    lax.fori_loop(0, 80, z1, 0)
    for kk in range(16):
        pltpu.sync_copy(zb2, aggrS.at[pl.ds(s * 1280 + kk * 80, 80)])
    pltpu.sync_copy(zb1, denS.at[pl.ds(s * 1280, 1280)])
    plsc.subcore_barrier()

    hv = jnp.broadcast_to(h, (16,))
    hhv = jnp.broadcast_to(hh, (16,))
    lane = lax.iota(jnp.int32, 16)
    cbase = chunk * ECHUNK

    def gblock(g, _):
        base = cbase + g * K
        pltpu.sync_copy(qidx_hbm.at[pl.ds(base, K)], qib)
        pltpu.sync_copy(kidx_hbm.at[pl.ds(base, K)], kib)
        pltpu.sync_copy(dst_hbm.at[pl.ds(base, K)], dib)

        for j in range(KB):
            def exbody(tt, _, j=j):
                o = j * 128 + tt * 16
                qv = qib[pl.ds(o, 16)]
                kv = kib[pl.ds(o, 16)]
                dv = dib[pl.ds(o, 16)]
                qi = plsc.load_gather(qh, [qv])
                kj = plsc.load_gather(kh, [kv])
                al = qi + kj
                al = jnp.where(al >= 0, al, al * NEG)
                exb[pl.ds(o, 16)] = jnp.exp(al)
                gidx[pl.ds(o, 16)] = kv * HEADS + hv
                sidx[j, pl.ds(tt * 16, 16)] = dv * 2 + hhv
                return 0
            lax.fori_loop(0, 8, exbody, 0)

        for j in range(KB):
            pltpu.sync_copy(xw_hbm.at[gidx.at[pl.ds(j * 128, 128)]],
                            rows.at[pl.ds(j * 128, 128)])

        def wbody(t, _):
            p = lane + t * 16
            e = lax.shift_right_logical(p, 5)
            cc = lax.bitwise_and(p, 31)
            w = plsc.load_gather(exb, [e])
            v = plsc.load_gather(rows, [e, cc])
            plsc.store_scatter(rows, [e, cc], v * w)
            return 0
        lax.fori_loop(0, K * OUT // 16, wbody, 0)

        for j in range(KB):
            pltpu.sync_copy(rows.at[pl.ds(j * 128, 128)],
                            aggrS.at[sidx.at[j]], add=True)
            pltpu.sync_copy(exb.at[pl.ds(j * 128, 128)],
                            denS.at[sidx.at[j]], add=True)
        return 0

    lax.fori_loop(0, NBLK, gblock, 0)
    plsc.subcore_barrier()

    pltpu.sync_copy(aggrS.at[pl.ds(s * 1280, 1280)],
                    aggr_out.at[c, pl.ds(s * 1280, 1280)])
    pltpu.sync_copy(denS.at[pl.ds(s * 1280, 1280)],
                    den_out.at[c, pl.ds(s * 1280, 1280)])


_sc_aggregate = functools.partial(
    pl.kernel,
    out_type=(
        jax.ShapeDtypeStruct((2, 2 * N2, OUT), jnp.float32),
        jax.ShapeDtypeStruct((2, 2 * N2), jnp.float32),
    ),
    mesh=plsc.VectorSubcoreMesh(core_axis_name="c", subcore_axis_name="s"),
    compiler_params=pltpu.CompilerParams(needs_layout_passes=False,
                                         use_tc_tiling_on_sc=False),
    scratch_types=[
        pltpu.VMEM((NREL * N,), jnp.float32),
        pltpu.VMEM((NREL * N,), jnp.float32),
        pltpu.VMEM((K,), jnp.int32),
        pltpu.VMEM((K,), jnp.int32),
        pltpu.VMEM((K,), jnp.int32),
        pltpu.VMEM((K,), jnp.int32),
        pltpu.VMEM((KB, 128), jnp.int32),
        pltpu.VMEM((K,), jnp.float32),
        pltpu.VMEM((K, OUT), jnp.float32),
        pltpu.VMEM((80, OUT), jnp.float32),
        pltpu.VMEM((1280,), jnp.float32),
        pltpu.VMEM_SHARED((2 * N2, OUT), jnp.float32),
        pltpu.VMEM_SHARED((2 * N2,), jnp.float32),
    ],
)(_sc_body)


def _layer(hin, qidx_p, kidx_p, dst_p, W, q, k, b, P, pb):
    xw, qn, kn = _tc_proj(hin, W, q, k)
    table = xw.reshape(NREL * N * HEADS, OUT)
    qnT = qn.transpose(2, 0, 1).reshape(HEADS, NREL * N)
    knT = kn.transpose(2, 0, 1).reshape(HEADS, NREL * N)
    aggr, den = _sc_aggregate(qidx_p, kidx_p, dst_p, qnT, knT, table)
    aggrv = aggr.reshape(2, N2, 2, OUT)
    denv = den.reshape(2, N2, 2)
    return _tc_finish(aggrv, denv, b.reshape(1, DIM), P, pb.reshape(1, DIM))


def kernel(x, eidx, etype, W0, q0, k0, b0, P0, pb0, W1, q1, k1, b1, P1, pb1):
    src = eidx[0]
    dst = eidx[1]
    qidx = etype * N + dst
    kidx = etype * N + src
    npad = EPAD - E
    zpad = jnp.zeros((npad,), jnp.int32)
    qidx_p = jnp.concatenate([qidx, zpad])
    kidx_p = jnp.concatenate([kidx, zpad])
    dst_p = jnp.concatenate([dst, jnp.full((npad,), N, jnp.int32)])
    h1 = _layer(x, qidx_p, kidx_p, dst_p, W0, q0, k0, b0, P0, pb0)
    return _layer(h1, qidx_p, kidx_p, dst_p, W1, q1, k1, b1, P1, pb1)


# speedup vs baseline: 52.9275x; 1.5109x over previous
import functools

import jax
import jax.numpy as jnp
from jax import lax
from jax.experimental import pallas as pl
from jax.experimental.pallas import tpu as pltpu
from jax.experimental.pallas import tpu_sc as plsc

N = 10000
E = 320000
DIM = 128
HEADS = 4
OUT = 32
NREL = 2
NEG = 0.2

N2 = 10240
EPAD = 327680
K = 1024
KB = K // 128
ECHUNK = EPAD // 8
NBLK = ECHUNK // K
BN = 1000


def _tc_proj(hin, W, q, k):
    grid = (NREL, N // BN)

    def body(h_ref, w_ref, q_ref, k_ref, xw_ref, qn_ref, kn_ref):
        xb = h_ref[...]
        xw = jnp.dot(xb, w_ref[0], preferred_element_type=jnp.float32)
        xw_ref[0] = xw
        qn_ref[0] = jnp.dot(xw, q_ref[...], preferred_element_type=jnp.float32)
        kn_ref[0] = jnp.dot(xw, k_ref[...], preferred_element_type=jnp.float32)

    return pl.pallas_call(
        body,
        grid=grid,
        in_specs=[
            pl.BlockSpec((BN, DIM), lambda r, nb: (nb, 0)),
            pl.BlockSpec((1, DIM, DIM), lambda r, nb: (r, 0, 0)),
            pl.BlockSpec((DIM, HEADS), lambda r, nb: (0, 0)),
            pl.BlockSpec((DIM, HEADS), lambda r, nb: (0, 0)),
        ],
        out_specs=[
            pl.BlockSpec((1, BN, DIM), lambda r, nb: (r, nb, 0)),
            pl.BlockSpec((1, BN, HEADS), lambda r, nb: (r, nb, 0)),
            pl.BlockSpec((1, BN, HEADS), lambda r, nb: (r, nb, 0)),
        ],
        out_shape=[
            jax.ShapeDtypeStruct((NREL, N, DIM), jnp.float32),
            jax.ShapeDtypeStruct((NREL, N, HEADS), jnp.float32),
            jax.ShapeDtypeStruct((NREL, N, HEADS), jnp.float32),
        ],
    )(hin, W, q, k)


def _tc_finish(aggr, den, b2, P, pb2):
    grid = (N // BN,)

    def body(a_ref, d_ref, b_ref, p_ref, pb_ref, o_ref):
        A = a_ref[...]
        d = d_ref[...]
        ag = jnp.concatenate(
            [A[h // 2, :, h % 2, :] for h in range(HEADS)], axis=1)
        den128 = jnp.concatenate(
            [jnp.broadcast_to(d[h // 2, :, h % 2:h % 2 + 1], (BN, OUT))
             for h in range(HEADS)], axis=1)
        feat = ag / (den128 + 1e-16) + b_ref[...]
        y = jnp.dot(feat, p_ref[...], preferred_element_type=jnp.float32)
        y = y + pb_ref[...]
        o_ref[...] = jnp.where(y > 0, y, jnp.exp(jnp.minimum(y, 0.0)) - 1.0)

    return pl.pallas_call(
        body,
        grid=grid,
        in_specs=[
            pl.BlockSpec((2, BN, 2, OUT), lambda nb: (0, nb, 0, 0)),
            pl.BlockSpec((2, BN, 2), lambda nb: (0, nb, 0)),
            pl.BlockSpec((1, DIM), lambda nb: (0, 0)),
            pl.BlockSpec((DIM, DIM), lambda nb: (0, 0)),
            pl.BlockSpec((1, DIM), lambda nb: (0, 0)),
        ],
        out_specs=pl.BlockSpec((BN, DIM), lambda nb: (nb, 0)),
        out_shape=jax.ShapeDtypeStruct((N, DIM), jnp.float32),
    )(aggr, den, b2, P, pb2)


def _sc_body(qidx_hbm, kidx_hbm, dst_hbm, qn_hbm, kn_hbm, xw_hbm,
             aggr_out, den_out,
             qh, kh, qib, kib, dib, gidx, sidx, exb, rows,
             zb2, zb1, aggrS, denS, sem):
    c = lax.axis_index("c")
    s = lax.axis_index("s")
    hh = lax.rem(s, 2)
    h = 2 * c + hh
    chunk = lax.div(s, 2)

    pltpu.sync_copy(qn_hbm.at[h], qh)
    pltpu.sync_copy(kn_hbm.at[h], kh)

    zv = jnp.zeros((16,), jnp.float32)
    for jr in range(80):
        zb2[jr, pl.ds(0, 16)] = zv
        zb2[jr, pl.ds(16, 16)] = zv

    def z1(i, _):
        zb1[pl.ds(i * 16, 16)] = zv
        return 0
    lax.fori_loop(0, 80, z1, 0)
    for kk in range(16):
        pltpu.sync_copy(zb2, aggrS.at[pl.ds(s * 1280 + kk * 80, 80)])
    pltpu.sync_copy(zb1, denS.at[pl.ds(s * 1280, 1280)])
    plsc.subcore_barrier()

    hv = jnp.broadcast_to(h, (16,))
    hhv = jnp.broadcast_to(hh, (16,))
    cbase = chunk * ECHUNK

    def gblock(g, _):
        base = cbase + g * K
        pltpu.sync_copy(qidx_hbm.at[pl.ds(base, K)], qib)
        pltpu.sync_copy(kidx_hbm.at[pl.ds(base, K)], kib)
        pltpu.sync_copy(dst_hbm.at[pl.ds(base, K)], dib)

        for j in range(KB):
            def ibody(tt, _, j=j):
                o = j * 128 + tt * 16
                kv = kib[pl.ds(o, 16)]
                dv = dib[pl.ds(o, 16)]
                gidx[pl.ds(o, 16)] = kv * HEADS + hv
                sidx[j, pl.ds(tt * 16, 16)] = dv * 2 + hhv
                return 0
            lax.fori_loop(0, 8, ibody, 0)

        handles = [
            pltpu.async_copy(xw_hbm.at[gidx.at[pl.ds(j * 128, 128)]],
                             rows.at[pl.ds(j * 128, 128)], sem)
            for j in range(KB)]

        for j in range(KB):
            def exbody(tt, _, j=j):
                o = j * 128 + tt * 16
                qv = qib[pl.ds(o, 16)]
                kv = kib[pl.ds(o, 16)]
                qi = plsc.load_gather(qh, [qv])
                kj = plsc.load_gather(kh, [kv])
                al = qi + kj
                al = jnp.where(al >= 0, al, al * NEG)
                exb[pl.ds(o, 16)] = jnp.exp(al)
                return 0
            lax.fori_loop(0, 8, exbody, 0)

        for hnd in handles:
            hnd.wait()

        def wbody(e, _):
            w = plsc.load_gather(exb, [jnp.broadcast_to(e, (16,))])
            rows[e, pl.ds(0, 16)] = rows[e, pl.ds(0, 16)] * w
            rows[e, pl.ds(16, 16)] = rows[e, pl.ds(16, 16)] * w
            return 0
        lax.fori_loop(0, K, wbody, 0)

        for j in range(KB):
            pltpu.sync_copy(rows.at[pl.ds(j * 128, 128)],
                            aggrS.at[sidx.at[j]], add=True)
            pltpu.sync_copy(exb.at[pl.ds(j * 128, 128)],
                            denS.at[sidx.at[j]], add=True)
        return 0

    lax.fori_loop(0, NBLK, gblock, 0)
    plsc.subcore_barrier()

    pltpu.sync_copy(aggrS.at[pl.ds(s * 1280, 1280)],
                    aggr_out.at[c, pl.ds(s * 1280, 1280)])
    pltpu.sync_copy(denS.at[pl.ds(s * 1280, 1280)],
                    den_out.at[c, pl.ds(s * 1280, 1280)])


_sc_aggregate = functools.partial(
    pl.kernel,
    out_type=(
        jax.ShapeDtypeStruct((2, 2 * N2, OUT), jnp.float32),
        jax.ShapeDtypeStruct((2, 2 * N2), jnp.float32),
    ),
    mesh=plsc.VectorSubcoreMesh(core_axis_name="c", subcore_axis_name="s"),
    compiler_params=pltpu.CompilerParams(needs_layout_passes=False,
                                         use_tc_tiling_on_sc=False),
    scratch_types=[
        pltpu.VMEM((NREL * N,), jnp.float32),
        pltpu.VMEM((NREL * N,), jnp.float32),
        pltpu.VMEM((K,), jnp.int32),
        pltpu.VMEM((K,), jnp.int32),
        pltpu.VMEM((K,), jnp.int32),
        pltpu.VMEM((K,), jnp.int32),
        pltpu.VMEM((KB, 128), jnp.int32),
        pltpu.VMEM((K,), jnp.float32),
        pltpu.VMEM((K, OUT), jnp.float32),
        pltpu.VMEM((80, OUT), jnp.float32),
        pltpu.VMEM((1280,), jnp.float32),
        pltpu.VMEM_SHARED((2 * N2, OUT), jnp.float32),
        pltpu.VMEM_SHARED((2 * N2,), jnp.float32),
        pltpu.SemaphoreType.DMA,
    ],
)(_sc_body)


def _layer(hin, qidx_p, kidx_p, dst_p, W, q, k, b, P, pb):
    xw, qn, kn = _tc_proj(hin, W, q, k)
    table = xw.reshape(NREL * N * HEADS, OUT)
    qnT = qn.transpose(2, 0, 1).reshape(HEADS, NREL * N)
    knT = kn.transpose(2, 0, 1).reshape(HEADS, NREL * N)
    aggr, den = _sc_aggregate(qidx_p, kidx_p, dst_p, qnT, knT, table)
    aggrv = aggr.reshape(2, N2, 2, OUT)
    denv = den.reshape(2, N2, 2)
    return _tc_finish(aggrv, denv, b.reshape(1, DIM), P, pb.reshape(1, DIM))


def kernel(x, eidx, etype, W0, q0, k0, b0, P0, pb0, W1, q1, k1, b1, P1, pb1):
    src = eidx[0]
    dst = eidx[1]
    qidx = etype * N + dst
    kidx = etype * N + src
    npad = EPAD - E
    zpad = jnp.zeros((npad,), jnp.int32)
    qidx_p = jnp.concatenate([qidx, zpad])
    kidx_p = jnp.concatenate([kidx, zpad])
    dst_p = jnp.concatenate([dst, jnp.full((npad,), N, jnp.int32)])
    h1 = _layer(x, qidx_p, kidx_p, dst_p, W0, q0, k0, b0, P0, pb0)
    return _layer(h1, qidx_p, kidx_p, dst_p, W1, q1, k1, b1, P1, pb1)


# speedup vs baseline: 67.7020x; 1.2791x over previous
import functools

import jax
import jax.numpy as jnp
from jax import lax
from jax.experimental import pallas as pl
from jax.experimental.pallas import tpu as pltpu
from jax.experimental.pallas import tpu_sc as plsc

N = 10000
E = 320000
DIM = 128
HEADS = 4
OUT = 32
NREL = 2
NEG = 0.2

N2 = 10240
EPAD = 327680
K = 1024
KB = K // 128
ECHUNK = EPAD // 8
NBLK = ECHUNK // K
BN = 1000


def _tc_proj(hin, W, q, k):
    grid = (NREL, N // BN)

    def body(h_ref, w_ref, q_ref, k_ref, xw_ref, qn_ref, kn_ref):
        xb = h_ref[...]
        xw = jnp.dot(xb, w_ref[0], preferred_element_type=jnp.float32)
        xw_ref[0] = xw
        qn_ref[0] = jnp.dot(xw, q_ref[...], preferred_element_type=jnp.float32)
        kn_ref[0] = jnp.dot(xw, k_ref[...], preferred_element_type=jnp.float32)

    return pl.pallas_call(
        body,
        grid=grid,
        in_specs=[
            pl.BlockSpec((BN, DIM), lambda r, nb: (nb, 0)),
            pl.BlockSpec((1, DIM, DIM), lambda r, nb: (r, 0, 0)),
            pl.BlockSpec((DIM, HEADS), lambda r, nb: (0, 0)),
            pl.BlockSpec((DIM, HEADS), lambda r, nb: (0, 0)),
        ],
        out_specs=[
            pl.BlockSpec((1, BN, DIM), lambda r, nb: (r, nb, 0)),
            pl.BlockSpec((1, BN, HEADS), lambda r, nb: (r, nb, 0)),
            pl.BlockSpec((1, BN, HEADS), lambda r, nb: (r, nb, 0)),
        ],
        out_shape=[
            jax.ShapeDtypeStruct((NREL, N, DIM), jnp.float32),
            jax.ShapeDtypeStruct((NREL, N, HEADS), jnp.float32),
            jax.ShapeDtypeStruct((NREL, N, HEADS), jnp.float32),
        ],
    )(hin, W, q, k)


def _tc_finish(aggr, den, b2, P, pb2):
    grid = (N // BN,)

    def body(a_ref, d_ref, b_ref, p_ref, pb_ref, o_ref):
        A = a_ref[...]
        d = d_ref[...]
        ag = jnp.concatenate(
            [A[h // 2, :, h % 2, :] for h in range(HEADS)], axis=1)
        den128 = jnp.concatenate(
            [jnp.broadcast_to(d[h // 2, :, h % 2:h % 2 + 1], (BN, OUT))
             for h in range(HEADS)], axis=1)
        feat = ag / (den128 + 1e-16) + b_ref[...]
        y = jnp.dot(feat, p_ref[...], preferred_element_type=jnp.float32)
        y = y + pb_ref[...]
        o_ref[...] = jnp.where(y > 0, y, jnp.exp(jnp.minimum(y, 0.0)) - 1.0)

    return pl.pallas_call(
        body,
        grid=grid,
        in_specs=[
            pl.BlockSpec((2, BN, 2, OUT), lambda nb: (0, nb, 0, 0)),
            pl.BlockSpec((2, BN, 2), lambda nb: (0, nb, 0)),
            pl.BlockSpec((1, DIM), lambda nb: (0, 0)),
            pl.BlockSpec((DIM, DIM), lambda nb: (0, 0)),
            pl.BlockSpec((1, DIM), lambda nb: (0, 0)),
        ],
        out_specs=pl.BlockSpec((BN, DIM), lambda nb: (nb, 0)),
        out_shape=jax.ShapeDtypeStruct((N, DIM), jnp.float32),
    )(aggr, den, b2, P, pb2)


def _sc_body(qidx_hbm, kidx_hbm, gidx_hbm, sidx_hbm, qn_hbm, kn_hbm, xw_hbm,
             aggr_out, den_out,
             qh, kh, qib, kib, gidx, sidx, exb, rows,
             zb2, zb1, aggrS, denS, sem):
    c = lax.axis_index("c")
    s = lax.axis_index("s")
    hh = lax.rem(s, 2)
    h = 2 * c + hh
    chunk = lax.div(s, 2)

    pltpu.sync_copy(qn_hbm.at[h], qh)
    pltpu.sync_copy(kn_hbm.at[h], kh)

    zv = jnp.zeros((16,), jnp.float32)
    for jr in range(80):
        zb2[jr, pl.ds(0, 16)] = zv
        zb2[jr, pl.ds(16, 16)] = zv

    def z1(i, _):
        zb1[pl.ds(i * 16, 16)] = zv
        return 0
    lax.fori_loop(0, 80, z1, 0)
    for kk in range(16):
        pltpu.sync_copy(zb2, aggrS.at[pl.ds(s * 1280 + kk * 80, 80)])
    pltpu.sync_copy(zb1, denS.at[pl.ds(s * 1280, 1280)])
    plsc.subcore_barrier()

    cbase = chunk * ECHUNK
    gbase = h * EPAD
    srow = hh * (EPAD // 128) + cbase // 128

    def gblock(g, _):
        base = cbase + g * K

        pltpu.sync_copy(gidx_hbm.at[pl.ds(gbase + base, K)], gidx)
        handles = [
            pltpu.async_copy(xw_hbm.at[gidx.at[pl.ds(j * 128, 128)]],
                             rows.at[pl.ds(j * 128, 128)], sem)
            for j in range(KB)]

        pltpu.sync_copy(sidx_hbm.at[pl.ds(srow + g * KB, KB)], sidx)
        pltpu.sync_copy(qidx_hbm.at[pl.ds(base, K)], qib)
        pltpu.sync_copy(kidx_hbm.at[pl.ds(base, K)], kib)

        @plsc.parallel_loop(0, K // 16, unroll=2)
        def exbody(tt):
            o = tt * 16
            qv = qib[pl.ds(o, 16)]
            kv = kib[pl.ds(o, 16)]
            qi = plsc.load_gather(qh, [qv])
            kj = plsc.load_gather(kh, [kv])
            al = qi + kj
            al = jnp.where(al >= 0, al, al * NEG)
            exb[pl.ds(o, 16)] = jnp.exp(al)

        for hnd in handles:
            hnd.wait()

        @plsc.parallel_loop(0, K, unroll=4)
        def wbody(e):
            w = plsc.load_gather(exb, [jnp.broadcast_to(e, (16,))])
            rows[e, pl.ds(0, 16)] = rows[e, pl.ds(0, 16)] * w
            rows[e, pl.ds(16, 16)] = rows[e, pl.ds(16, 16)] * w

        for j in range(KB):
            pltpu.sync_copy(rows.at[pl.ds(j * 128, 128)],
                            aggrS.at[sidx.at[j]], add=True)
            pltpu.sync_copy(exb.at[pl.ds(j * 128, 128)],
                            denS.at[sidx.at[j]], add=True)
        return 0

    lax.fori_loop(0, NBLK, gblock, 0)
    plsc.subcore_barrier()

    pltpu.sync_copy(aggrS.at[pl.ds(s * 1280, 1280)],
                    aggr_out.at[c, pl.ds(s * 1280, 1280)])
    pltpu.sync_copy(denS.at[pl.ds(s * 1280, 1280)],
                    den_out.at[c, pl.ds(s * 1280, 1280)])


_sc_aggregate = functools.partial(
    pl.kernel,
    out_type=(
        jax.ShapeDtypeStruct((2, 2 * N2, OUT), jnp.float32),
        jax.ShapeDtypeStruct((2, 2 * N2), jnp.float32),
    ),
    mesh=plsc.VectorSubcoreMesh(core_axis_name="c", subcore_axis_name="s"),
    compiler_params=pltpu.CompilerParams(needs_layout_passes=False,
                                         use_tc_tiling_on_sc=False),
    scratch_types=[
        pltpu.VMEM((NREL * N,), jnp.float32),
        pltpu.VMEM((NREL * N,), jnp.float32),
        pltpu.VMEM((K,), jnp.int32),
        pltpu.VMEM((K,), jnp.int32),
        pltpu.VMEM((K,), jnp.int32),
        pltpu.VMEM((KB, 128), jnp.int32),
        pltpu.VMEM((K,), jnp.float32),
        pltpu.VMEM((K, OUT), jnp.float32),
        pltpu.VMEM((80, OUT), jnp.float32),
        pltpu.VMEM((1280,), jnp.float32),
        pltpu.VMEM_SHARED((2 * N2, OUT), jnp.float32),
        pltpu.VMEM_SHARED((2 * N2,), jnp.float32),
        pltpu.SemaphoreType.DMA,
    ],
)(_sc_body)


def _layer(hin, qidx_p, kidx_p, gidx_all, sidx_all, W, q, k, b, P, pb):
    xw, qn, kn = _tc_proj(hin, W, q, k)
    table = xw.reshape(NREL * N * HEADS, OUT)
    qnT = qn.transpose(2, 0, 1).reshape(HEADS, NREL * N)
    knT = kn.transpose(2, 0, 1).reshape(HEADS, NREL * N)
    aggr, den = _sc_aggregate(qidx_p, kidx_p, gidx_all, sidx_all, qnT, knT,
                              table)
    aggrv = aggr.reshape(2, N2, 2, OUT)
    denv = den.reshape(2, N2, 2)
    return _tc_finish(aggrv, denv, b.reshape(1, DIM), P, pb.reshape(1, DIM))


def kernel(x, eidx, etype, W0, q0, k0, b0, P0, pb0, W1, q1, k1, b1, P1, pb1):
    src = eidx[0]
    dst = eidx[1]
    qidx = etype * N + dst
    kidx = etype * N + src
    npad = EPAD - E
    zpad = jnp.zeros((npad,), jnp.int32)
    qidx_p = jnp.concatenate([qidx, zpad])
    kidx_p = jnp.concatenate([kidx, zpad])
    dst_p = jnp.concatenate([dst, jnp.full((npad,), N, jnp.int32)])
    harr = jnp.arange(HEADS, dtype=jnp.int32)
    gidx_all = (kidx_p[None, :] * HEADS + harr[:, None]).reshape(-1)
    sidx_all = (dst_p[None, :] * 2
                + jnp.arange(2, dtype=jnp.int32)[:, None]
                ).reshape(2 * EPAD // 128, 128)
    h1 = _layer(x, qidx_p, kidx_p, gidx_all, sidx_all, W0, q0, k0, b0, P0, pb0)
    return _layer(h1, qidx_p, kidx_p, gidx_all, sidx_all, W1, q1, k1, b1, P1,
                  pb1)


# speedup vs baseline: 75.0764x; 1.1089x over previous
import functools

import jax
import jax.numpy as jnp
from jax import lax
from jax.experimental import pallas as pl
from jax.experimental.pallas import tpu as pltpu
from jax.experimental.pallas import tpu_sc as plsc

N = 10000
E = 320000
DIM = 128
HEADS = 4
OUT = 32
NREL = 2
NEG = 0.2

N2 = 10240
EPAD = 327680
K = 1024
KB = K // 128
ECHUNK = EPAD // 8
NBLK = ECHUNK // K
BN = 1000


def _tc_proj(hin, W, q, k):
    grid = (NREL, N // BN)

    def body(h_ref, w_ref, q_ref, k_ref, xw_ref, qn_ref, kn_ref):
        xb = h_ref[...]
        xw = jnp.dot(xb, w_ref[0], preferred_element_type=jnp.float32)
        xw_ref[0] = xw
        qn_ref[0] = jnp.dot(xw, q_ref[...], preferred_element_type=jnp.float32)
        kn_ref[0] = jnp.dot(xw, k_ref[...], preferred_element_type=jnp.float32)

    return pl.pallas_call(
        body,
        grid=grid,
        in_specs=[
            pl.BlockSpec((BN, DIM), lambda r, nb: (nb, 0)),
            pl.BlockSpec((1, DIM, DIM), lambda r, nb: (r, 0, 0)),
            pl.BlockSpec((DIM, HEADS), lambda r, nb: (0, 0)),
            pl.BlockSpec((DIM, HEADS), lambda r, nb: (0, 0)),
        ],
        out_specs=[
            pl.BlockSpec((1, BN, DIM), lambda r, nb: (r, nb, 0)),
            pl.BlockSpec((1, BN, HEADS), lambda r, nb: (r, nb, 0)),
            pl.BlockSpec((1, BN, HEADS), lambda r, nb: (r, nb, 0)),
        ],
        out_shape=[
            jax.ShapeDtypeStruct((NREL, N, DIM), jnp.float32),
            jax.ShapeDtypeStruct((NREL, N, HEADS), jnp.float32),
            jax.ShapeDtypeStruct((NREL, N, HEADS), jnp.float32),
        ],
    )(hin, W, q, k)


def _tc_finish(aggr, den, b2, P, pb2):
    grid = (N // BN,)

    def body(a_ref, d_ref, b_ref, p_ref, pb_ref, o_ref):
        A = a_ref[...]
        d = d_ref[...]
        ag = jnp.concatenate(
            [A[h // 2, :, h % 2, :] for h in range(HEADS)], axis=1)
        den128 = jnp.concatenate(
            [jnp.broadcast_to(d[h // 2, :, h % 2:h % 2 + 1], (BN, OUT))
             for h in range(HEADS)], axis=1)
        feat = ag / (den128 + 1e-16) + b_ref[...]
        y = jnp.dot(feat, p_ref[...], preferred_element_type=jnp.float32)
        y = y + pb_ref[...]
        o_ref[...] = jnp.where(y > 0, y, jnp.exp(jnp.minimum(y, 0.0)) - 1.0)

    return pl.pallas_call(
        body,
        grid=grid,
        in_specs=[
            pl.BlockSpec((2, BN, 2, OUT), lambda nb: (0, nb, 0, 0)),
            pl.BlockSpec((2, BN, 2), lambda nb: (0, nb, 0)),
            pl.BlockSpec((1, DIM), lambda nb: (0, 0)),
            pl.BlockSpec((DIM, DIM), lambda nb: (0, 0)),
            pl.BlockSpec((1, DIM), lambda nb: (0, 0)),
        ],
        out_specs=pl.BlockSpec((BN, DIM), lambda nb: (nb, 0)),
        out_shape=jax.ShapeDtypeStruct((N, DIM), jnp.float32),
    )(aggr, den, b2, P, pb2)


def _sc_body(gidx_hbm, qgidx_hbm, sidx_hbm, qn_hbm, kn_hbm, xw_hbm,
             aggr_out, den_out,
             gx0, gx1, qgx0, qgx1, sx0, sx1, qv0, qv1, kv0, kv1,
             ex0, ex1, rw0, rw1,
             zb2, zb1, aggrS, denS, sg0, sg1, ss0, ss1, si0, si1):
    c = lax.axis_index("c")
    s = lax.axis_index("s")
    hh = lax.rem(s, 2)
    h = 2 * c + hh
    chunk = lax.div(s, 2)
    gx = (gx0, gx1)
    qgx = (qgx0, qgx1)
    sx = (sx0, sx1)
    qv = (qv0, qv1)
    kv = (kv0, kv1)
    ex = (ex0, ex1)
    rw = (rw0, rw1)
    sg = (sg0, sg1)
    ss = (ss0, ss1)
    si = (si0, si1)

    zv = jnp.zeros((16,), jnp.float32)
    for jr in range(80):
        zb2[jr, pl.ds(0, 16)] = zv
        zb2[jr, pl.ds(16, 16)] = zv

    def z1(i, _):
        zb1[pl.ds(i * 16, 16)] = zv
        return 0
    lax.fori_loop(0, 80, z1, 0)
    for kk in range(16):
        pltpu.sync_copy(zb2, aggrS.at[pl.ds(s * 1280 + kk * 80, 80)])
    pltpu.sync_copy(zb1, denS.at[pl.ds(s * 1280, 1280)])
    plsc.subcore_barrier()

    cbase = chunk * ECHUNK
    gbase = h * EPAD
    srow = hh * (EPAD // 128) + cbase // 128

    def idx_copies(g, b, mk):
        base = cbase + g * K
        return [
            mk(gidx_hbm.at[pl.ds(gbase + base, K)], gx[b], si[b]),
            mk(qgidx_hbm.at[pl.ds(gbase + base, K)], qgx[b], si[b]),
            mk(sidx_hbm.at[pl.ds(srow + g * KB, KB)], sx[b], si[b]),
        ]

    def gather_copies(b, mk):
        hs = []
        for j in range(KB):
            sl = pl.ds(j * 128, 128)
            hs.append(mk(xw_hbm.at[gx[b].at[sl]], rw[b].at[sl], sg[b]))
            hs.append(mk(qn_hbm.at[qgx[b].at[sl]], qv[b].at[sl], sg[b]))
            hs.append(mk(kn_hbm.at[gx[b].at[sl]], kv[b].at[sl], sg[b]))
        return hs

    def drain(copies):
        for hnd in copies:
            hnd.wait()

    def fire_scatters(b):
        hs = []
        for j in range(KB):
            sl = pl.ds(j * 128, 128)
            hs.append(pltpu.async_copy(rw[b].at[sl],
                                       aggrS.at[sx[b].at[j]], ss0, add=True))
            hs.append(pltpu.async_copy(ex[b].at[sl],
                                       denS.at[sx[b].at[j]], ss0, add=True))
        return hs

    def run_exbody(b):
        @plsc.parallel_loop(0, K // 16, unroll=2)
        def exbody(tt):
            o = tt * 16
            al = qv[b][pl.ds(o, 16)] + kv[b][pl.ds(o, 16)]
            al = jnp.where(al >= 0, al, al * NEG)
            ex[b][pl.ds(o, 16)] = jnp.exp(al)

    def run_wbody(b):
        @plsc.parallel_loop(0, K, unroll=4)
        def wbody(e):
            w = plsc.load_gather(ex[b], [jnp.broadcast_to(e, (16,))])
            rw[b][e, pl.ds(0, 16)] = rw[b][e, pl.ds(0, 16)] * w
            rw[b][e, pl.ds(16, 16)] = rw[b][e, pl.ds(16, 16)] * w

    def block(g, b, has_next, has_next2):
        nb = 1 - b
        drain(gather_copies(b, pltpu.make_async_copy))
        run_exbody(b)

        def prep_next():
            drain(idx_copies(g + 1, nb, pltpu.make_async_copy))
            gather_copies(nb, pltpu.async_copy)
        if has_next is True:
            prep_next()
        else:
            pl.when(has_next)(prep_next)

        run_wbody(b)
        drain(fire_scatters(b))

        def prep_next2():
            idx_copies(g + 2, b, pltpu.async_copy)
        if has_next2 is True:
            prep_next2()
        else:
            pl.when(has_next2)(prep_next2)

    drain(idx_copies(0, 0, pltpu.async_copy))
    gather_copies(0, pltpu.async_copy)
    idx_copies(1, 1, pltpu.async_copy)

    def pair(i, _):
        g0 = 2 * i
        block(g0, 0, True, i < NBLK // 2 - 1)
        block(g0 + 1, 1, i < NBLK // 2 - 1, i < NBLK // 2 - 1)
        return 0

    lax.fori_loop(0, NBLK // 2, pair, 0)
    plsc.subcore_barrier()

    pltpu.sync_copy(aggrS.at[pl.ds(s * 1280, 1280)],
                    aggr_out.at[c, pl.ds(s * 1280, 1280)])
    pltpu.sync_copy(denS.at[pl.ds(s * 1280, 1280)],
                    den_out.at[c, pl.ds(s * 1280, 1280)])


_sc_aggregate = functools.partial(
    pl.kernel,
    out_type=(
        jax.ShapeDtypeStruct((2, 2 * N2, OUT), jnp.float32),
        jax.ShapeDtypeStruct((2, 2 * N2), jnp.float32),
    ),
    mesh=plsc.VectorSubcoreMesh(core_axis_name="c", subcore_axis_name="s"),
    compiler_params=pltpu.CompilerParams(needs_layout_passes=False,
                                         use_tc_tiling_on_sc=False),
    scratch_types=[
        pltpu.VMEM((K,), jnp.int32),
        pltpu.VMEM((K,), jnp.int32),
        pltpu.VMEM((K,), jnp.int32),
        pltpu.VMEM((K,), jnp.int32),
        pltpu.VMEM((KB, 128), jnp.int32),
        pltpu.VMEM((KB, 128), jnp.int32),
        pltpu.VMEM((K,), jnp.float32),
        pltpu.VMEM((K,), jnp.float32),
        pltpu.VMEM((K,), jnp.float32),
        pltpu.VMEM((K,), jnp.float32),
        pltpu.VMEM((K,), jnp.float32),
        pltpu.VMEM((K,), jnp.float32),
        pltpu.VMEM((K, OUT), jnp.float32),
        pltpu.VMEM((K, OUT), jnp.float32),
        pltpu.VMEM((80, OUT), jnp.float32),
        pltpu.VMEM((1280,), jnp.float32),
        pltpu.VMEM_SHARED((2 * N2, OUT), jnp.float32),
        pltpu.VMEM_SHARED((2 * N2,), jnp.float32),
        pltpu.SemaphoreType.DMA,
        pltpu.SemaphoreType.DMA,
        pltpu.SemaphoreType.DMA,
        pltpu.SemaphoreType.DMA,
        pltpu.SemaphoreType.DMA,
        pltpu.SemaphoreType.DMA,
    ],
)(_sc_body)


def _layer(hin, gidx_all, qgidx_all, sidx_all, W, q, k, b, P, pb):
    xw, qn, kn = _tc_proj(hin, W, q, k)
    table = xw.reshape(NREL * N * HEADS, OUT)
    aggr, den = _sc_aggregate(gidx_all, qgidx_all, sidx_all,
                              qn.reshape(-1), kn.reshape(-1), table)
    aggrv = aggr.reshape(2, N2, 2, OUT)
    denv = den.reshape(2, N2, 2)
    return _tc_finish(aggrv, denv, b.reshape(1, DIM), P, pb.reshape(1, DIM))


def kernel(x, eidx, etype, W0, q0, k0, b0, P0, pb0, W1, q1, k1, b1, P1, pb1):
    src = eidx[0]
    dst = eidx[1]
    qidx = etype * N + dst
    kidx = etype * N + src
    npad = EPAD - E
    zpad = jnp.zeros((npad,), jnp.int32)
    qidx_p = jnp.concatenate([qidx, zpad])
    kidx_p = jnp.concatenate([kidx, zpad])
    dst_p = jnp.concatenate([dst, jnp.full((npad,), N, jnp.int32)])
    harr = jnp.arange(HEADS, dtype=jnp.int32)
    gidx_all = (kidx_p[None, :] * HEADS + harr[:, None]).reshape(-1)
    qgidx_all = (qidx_p[None, :] * HEADS + harr[:, None]).reshape(-1)
    sidx_all = (dst_p[None, :] * 2
                + jnp.arange(2, dtype=jnp.int32)[:, None]
                ).reshape(2 * EPAD // 128, 128)
    h1 = _layer(x, gidx_all, qgidx_all, sidx_all, W0, q0, k0, b0, P0, pb0)
    return _layer(h1, gidx_all, qgidx_all, sidx_all, W1, q1, k1, b1, P1, pb1)
